# Initial kernel scaffold; baseline (speedup 1.0000x reference)
import jax
import jax.numpy as jnp
from jax.experimental import pallas as pl


def kernel(x):
    raise NotImplementedError("write your pallas kernel here")


# speedup vs baseline: 215.5371x; 215.5371x over previous
import functools

import jax
import jax.numpy as jnp
from jax import lax
from jax.experimental import pallas as pl
from jax.experimental.pallas import tpu as pltpu
from jax.experimental.pallas import tpu_sc as plsc

IN_H = IN_W = 512
OUT_H = OUT_W = 224
BATCH = 64
NUMEL = IN_H * IN_W
TOTAL = BATCH * NUMEL
K_IDX = int(0.1 * TOTAL)
CLIP_LO = 1e-3
NBINS = 1 << 16
HPAD = NBINS + 16
QN = OUT_H * OUT_W

NC, NS, LANES = 2, 16, 16
NW = NC * NS
ROWS_PER_W = BATCH // NW
CHUNK = 4096
QCHUNK = 6272

_INT_MIN = -2147483648


def _key16(xv):
    ib = plsc.bitcast(xv, jnp.int32)
    key = jnp.where(ib >= 0, ib ^ jnp.int32(_INT_MIN), ~ib)
    hi = (key >> 16) & jnp.int32(0xFFFF)
    lo = key & jnp.int32(0xFFFF)
    return hi, lo


def _wid():
    return lax.axis_index("s") * NC + lax.axis_index("c")


def _make_kernels(interpret=False):
    mesh = plsc.VectorSubcoreMesh(core_axis_name="c", subcore_axis_name="s")

    @functools.partial(
        pl.kernel,
        out_type=[
            jax.ShapeDtypeStruct((BATCH, HPAD), jnp.int32),
            jax.ShapeDtypeStruct((NW, LANES), jnp.int32),
        ],
        mesh=mesh,
        scratch_types=[
            pltpu.VMEM((HPAD,), jnp.int32),
            pltpu.VMEM((CHUNK,), jnp.float32),
            pltpu.VMEM((LANES,), jnp.int32),
        ],
        interpret=interpret,
        compiler_params=pltpu.CompilerParams(needs_layout_passes=False),
    )
    def hist_kernel(x_hbm, cdf_hbm, cnt_hbm, hist_v, buf_v, cnt_v):
        wid = _wid()
        ones = jnp.ones((LANES,), jnp.int32)

        def row_pass(j, cnt):
            row = wid * ROWS_PER_W + j

            def zero_body(i, _):
                hist_v[pl.ds(i * LANES, LANES)] = jnp.zeros((LANES,), jnp.int32)
                return 0
            lax.fori_loop(0, HPAD // LANES, zero_body, 0)

            def chunk_body(c, cnt):
                pltpu.sync_copy(x_hbm.at[row, pl.ds(c * CHUNK, CHUNK)], buf_v)

                def vec_body(v, cnt):
                    xv = buf_v[pl.ds(v * LANES, LANES)]
                    hi, _ = _key16(xv)
                    plsc.addupdate_scatter(hist_v, [hi], ones)
                    return cnt + jnp.where(xv <= jnp.float32(CLIP_LO), 1, 0).astype(jnp.int32)
                return lax.fori_loop(0, CHUNK // LANES, vec_body, cnt)
            cnt = lax.fori_loop(0, NUMEL // CHUNK, chunk_body, cnt)

            def cum_body(i, run):
                h = hist_v[pl.ds(i * LANES, LANES)]
                hist_v[pl.ds(i * LANES, LANES)] = plsc.cumsum(h) - h + run
                return run + jnp.sum(h)
            run = lax.fori_loop(0, NBINS // LANES, cum_body, jnp.int32(0))
            hist_v[pl.ds(NBINS, LANES)] = jnp.zeros((LANES,), jnp.int32) + run

            pltpu.sync_copy(hist_v, cdf_hbm.at[row])
            return cnt

        cnt = jnp.zeros((LANES,), jnp.int32)
        cnt = row_pass(0, cnt)
        cnt = row_pass(1, cnt)
        cnt_v[...] = cnt
        pltpu.sync_copy(cnt_v, cnt_hbm.at[wid])

    @functools.partial(
        pl.kernel,
        out_type=jax.ShapeDtypeStruct((BATCH, QN), jnp.float32),
        mesh=mesh,
        scratch_types=[
            pltpu.VMEM((HPAD,), jnp.int32),
            pltpu.VMEM((QCHUNK,), jnp.float32),
            pltpu.VMEM((QCHUNK,), jnp.float32),
            pltpu.VMEM((LANES,), jnp.float32),
        ],
        interpret=interpret,
        compiler_params=pltpu.CompilerParams(needs_layout_passes=False),
    )
    def rank_kernel(q_hbm, cdf_hbm, t_hbm, out_hbm, cdf_v, qbuf_v, obuf_v, t_v):
        wid = _wid()
        pltpu.sync_copy(t_hbm, t_v)
        tv = t_v[...]
        inv_bin = jnp.float32(1.0 / 65536.0)
        inv_n = jnp.float32(1.0 / NUMEL)

        def row_pass(j, _):
            row = wid * ROWS_PER_W + j
            pltpu.sync_copy(cdf_hbm.at[row], cdf_v)

            def chunk_body(c, _):
                pltpu.sync_copy(q_hbm.at[row, pl.ds(c * QCHUNK, QCHUNK)], qbuf_v)

                def vec_body(v, _):
                    xv = qbuf_v[pl.ds(v * LANES, LANES)]
                    hi, lo = _key16(xv)
                    g0 = plsc.load_gather(cdf_v, [hi]).astype(jnp.float32)
                    g1 = plsc.load_gather(cdf_v, [hi + 1]).astype(jnp.float32)
                    rank = g0 + (g1 - g0) * (lo.astype(jnp.float32) * inv_bin)
                    obuf_v[pl.ds(v * LANES, LANES)] = jnp.where(
                        xv > tv, rank * inv_n, jnp.float32(0.0))
                    return 0
                lax.fori_loop(0, QCHUNK // LANES, vec_body, 0)
                pltpu.sync_copy(obuf_v, out_hbm.at[row, pl.ds(c * QCHUNK, QCHUNK)])
                return 0
            lax.fori_loop(0, QN // QCHUNK, chunk_body, 0)
            return 0

        row_pass(0, 0)
        row_pass(1, 0)

    @functools.partial(
        pl.kernel,
        out_type=jax.ShapeDtypeStruct((NW, NBINS), jnp.int32),
        mesh=mesh,
        scratch_types=[
            pltpu.VMEM((NBINS,), jnp.int32),
            pltpu.VMEM((CHUNK,), jnp.float32),
            pltpu.VMEM((LANES,), jnp.int32),
        ],
        interpret=interpret,
        compiler_params=pltpu.CompilerParams(needs_layout_passes=False),
    )
    def lohist_kernel(x_hbm, bstar_hbm, lh_hbm, hist_v, buf_v, b_v):
        wid = _wid()
        pltpu.sync_copy(bstar_hbm, b_v)
        bstar = b_v[...]
        ones = jnp.ones((LANES,), jnp.int32)

        def zero_body(i, _):
            hist_v[pl.ds(i * LANES, LANES)] = jnp.zeros((LANES,), jnp.int32)
            return 0
        lax.fori_loop(0, NBINS // LANES, zero_body, 0)

        def row_pass(j, _):
            row = wid * ROWS_PER_W + j

            def chunk_body(c, _):
                pltpu.sync_copy(x_hbm.at[row, pl.ds(c * CHUNK, CHUNK)], buf_v)

                def vec_body(v, _):
                    xv = buf_v[pl.ds(v * LANES, LANES)]
                    hi, lo = _key16(xv)
                    plsc.addupdate_scatter(hist_v, [lo], ones, mask=hi == bstar)
                    return 0
                lax.fori_loop(0, CHUNK // LANES, vec_body, 0)
                return 0
            lax.fori_loop(0, NUMEL // CHUNK, chunk_body, 0)
            return 0

        row_pass(0, 0)
        row_pass(1, 0)
        pltpu.sync_copy(hist_v, lh_hbm.at[wid])

    return hist_kernel, rank_kernel, lohist_kernel


_hist_kernel, _rank_kernel, _lohist_kernel = _make_kernels()


def _exact_eps(xf, cdf):
    gcum = jnp.sum(cdf[:, : NBINS + 1], axis=0)
    bstar = jnp.searchsorted(gcum, K_IDX, side="right").astype(jnp.int32) - 1
    rstar = K_IDX - gcum[bstar]
    lh = jnp.sum(_lohist_kernel(xf, jnp.full((LANES,), bstar, jnp.int32)), axis=0)
    lc = jnp.cumsum(lh)
    lostar = jnp.searchsorted(lc, rstar, side="right").astype(jnp.int32)
    key = (bstar << 16) | lostar
    orig = jnp.where(key < 0, key ^ jnp.int32(_INT_MIN), ~key)
    eps = lax.bitcast_convert_type(orig, jnp.float32)
    return jnp.maximum(eps, jnp.float32(CLIP_LO))


def kernel(x):
    xf = x.reshape(BATCH, NUMEL)
    cdf, counts = _hist_kernel(xf)
    c = jnp.sum(counts)
    t = lax.cond(c > K_IDX, lambda: jnp.float32(CLIP_LO), lambda: _exact_eps(xf, cdf))

    h_idx = (jnp.arange(OUT_H) * IN_H) // OUT_H
    w_idx = (jnp.arange(OUT_W) * IN_W) // OUT_W
    q = x.reshape(BATCH, IN_H, IN_W)[:, h_idx][:, :, w_idx].reshape(BATCH, QN)

    out = _rank_kernel(q, cdf, jnp.full((LANES,), t, jnp.float32))
    return out.reshape(BATCH, OUT_H, OUT_W)


# speedup vs baseline: 282.5116x; 1.3107x over previous
import functools

import jax
import jax.numpy as jnp
from jax import lax
from jax.experimental import pallas as pl
from jax.experimental.pallas import tpu as pltpu
from jax.experimental.pallas import tpu_sc as plsc

IN_H = IN_W = 512
OUT_H = OUT_W = 224
BATCH = 64
NUMEL = IN_H * IN_W
TOTAL = BATCH * NUMEL
K_IDX = int(0.1 * TOTAL)
CLIP_LO = 1e-3
NBINS = 1 << 16
HPAD = NBINS + 16
QN = OUT_H * OUT_W

NC, NS, LANES = 2, 16, 16
NW = NC * NS
ROWS_PER_W = BATCH // NW
CHUNK = 8192
NCHUNK = NUMEL // CHUNK
QCHUNK = 6272
NQCHUNK = QN // QCHUNK
NSEG = 16
SEGBINS = NBINS // NSEG

_INT_MIN = -2147483648


def _wid():
    return lax.axis_index("s") * NC + lax.axis_index("c")


def _key_full(ib):
    key = ib ^ ((ib >> 31) | jnp.int32(_INT_MIN))
    hi = (key >> 16) & jnp.int32(0xFFFF)
    lo = key & jnp.int32(0xFFFF)
    return hi, lo


def _key_hi(ib):
    s = ib >> 16
    return s ^ ((s >> 31) | jnp.int32(0x8000))


def _make_kernels(interpret=False):
    mesh = plsc.VectorSubcoreMesh(core_axis_name="c", subcore_axis_name="s")

    @functools.partial(
        pl.kernel,
        out_type=[
            jax.ShapeDtypeStruct((BATCH, QN), jnp.float32),
            jax.ShapeDtypeStruct((BATCH, HPAD), jnp.int32),
            jax.ShapeDtypeStruct((NW, LANES), jnp.int32),
        ],
        mesh=mesh,
        scratch_types=[
            pltpu.VMEM((HPAD,), jnp.int32),
            pltpu.VMEM((2, CHUNK), jnp.float32),
            pltpu.VMEM((2, QCHUNK), jnp.float32),
            pltpu.VMEM((2, QCHUNK), jnp.float32),
            pltpu.VMEM((LANES,), jnp.int32),
            pltpu.SemaphoreType.DMA,
            pltpu.SemaphoreType.DMA,
            pltpu.SemaphoreType.DMA,
            pltpu.SemaphoreType.DMA,
            pltpu.SemaphoreType.DMA,
            pltpu.SemaphoreType.DMA,
        ],
        interpret=interpret,
        compiler_params=pltpu.CompilerParams(needs_layout_passes=False),
    )
    def main_kernel(x_hbm, q_hbm, rk_hbm, cdf_hbm, cnt_hbm,
                    hist_v, xbuf_v, qbuf_v, obuf_v, cnt_v,
                    sx0, sx1, sq0, sq1, so0, so1):
        wid = _wid()
        sx = (sx0, sx1)
        sq = (sq0, sq1)
        so = (so0, so1)
        ones = jnp.ones((LANES,), jnp.int32)
        zeros = jnp.zeros((LANES,), jnp.int32)
        clip = jnp.float32(CLIP_LO)
        inv_bin = jnp.float32(1.0 / 65536.0)
        inv_n = jnp.float32(1.0 / NUMEL)

        def row_pass(j, cnt):
            row = wid * ROWS_PER_W + j

            def zero_body(i, _):
                for k in range(4):
                    hist_v[pl.ds(i * 64 + k * LANES, LANES)] = zeros
                return 0
            lax.fori_loop(0, NBINS // 64, zero_body, 0)
            hist_v[pl.ds(NBINS, LANES)] = zeros

            pltpu.async_copy(x_hbm.at[row, pl.ds(0, CHUNK)], xbuf_v.at[0], sx[0])

            def chunk_pair(cc, cnt):
                for b in range(2):
                    ch = cc * 2 + b
                    pltpu.make_async_copy(
                        x_hbm.at[row, pl.ds(ch * CHUNK, CHUNK)],
                        xbuf_v.at[b], sx[b]).wait()
                    if b == 0:
                        pltpu.async_copy(
                            x_hbm.at[row, pl.ds((ch + 1) * CHUNK, CHUNK)],
                            xbuf_v.at[1], sx[1])
                    else:
                        @pl.when(cc < NCHUNK // 2 - 1)
                        def _():
                            pltpu.async_copy(
                                x_hbm.at[row, pl.ds((ch + 1) * CHUNK, CHUNK)],
                                xbuf_v.at[0], sx[0])

                    def vec_body(v, cnt):
                        for k in range(4):
                            xv = xbuf_v[b, pl.ds(v * 64 + k * LANES, LANES)]
                            hi = _key_hi(plsc.bitcast(xv, jnp.int32))
                            plsc.addupdate_scatter(hist_v, [hi], ones)
                            cnt = cnt + jnp.where(xv <= clip, 1, 0).astype(jnp.int32)
                        return cnt
                    cnt = lax.fori_loop(0, CHUNK // 64, vec_body, cnt)
                return cnt
            cnt = lax.fori_loop(0, NCHUNK // 2, chunk_pair, cnt)

            seg_tot = []
            for s in range(NSEG):
                def acc_body(i, a, s=s):
                    a0, a1, a2, a3 = a
                    base = s * SEGBINS + i * 64
                    a0 = a0 + hist_v[pl.ds(base, LANES)]
                    a1 = a1 + hist_v[pl.ds(base + LANES, LANES)]
                    a2 = a2 + hist_v[pl.ds(base + 2 * LANES, LANES)]
                    a3 = a3 + hist_v[pl.ds(base + 3 * LANES, LANES)]
                    return (a0, a1, a2, a3)
                a = lax.fori_loop(0, SEGBINS // 64, acc_body,
                                  (zeros, zeros, zeros, zeros))
                seg_tot.append(jnp.sum(a[0] + a[1] + a[2] + a[3]))

            seg_base = []
            run = jnp.int32(0)
            for s in range(NSEG):
                seg_base.append(run)
                run = run + seg_tot[s]

            def cum_body(i, runs):
                new_runs = []
                for s in range(NSEG):
                    off = s * SEGBINS + i * LANES
                    h = hist_v[pl.ds(off, LANES)]
                    hist_v[pl.ds(off, LANES)] = plsc.cumsum(h) - h + runs[s]
                    new_runs.append(runs[s] + jnp.sum(h))
                return tuple(new_runs)
            lax.fori_loop(0, SEGBINS // LANES, cum_body, tuple(seg_base))
            hist_v[pl.ds(NBINS, LANES)] = zeros + run

            pltpu.sync_copy(hist_v, cdf_hbm.at[row])

            pltpu.async_copy(q_hbm.at[row, pl.ds(0, QCHUNK)], qbuf_v.at[0], sq[0])

            def q_pair(cc, _):
                for b in range(2):
                    ch = cc * 2 + b
                    pltpu.make_async_copy(
                        q_hbm.at[row, pl.ds(ch * QCHUNK, QCHUNK)],
                        qbuf_v.at[b], sq[b]).wait()
                    if b == 0:
                        pltpu.async_copy(
                            q_hbm.at[row, pl.ds((ch + 1) * QCHUNK, QCHUNK)],
                            qbuf_v.at[1], sq[1])
                    else:
                        @pl.when(cc < NQCHUNK // 2 - 1)
                        def _():
                            pltpu.async_copy(
                                q_hbm.at[row, pl.ds((ch + 1) * QCHUNK, QCHUNK)],
                                qbuf_v.at[0], sq[0])

                    @pl.when(cc > 0)
                    def _():
                        pltpu.make_async_copy(
                            obuf_v.at[b],
                            rk_hbm.at[row, pl.ds((ch - 2) * QCHUNK, QCHUNK)],
                            so[b]).wait()

                    def vec_body(v, _):
                        for k in range(4):
                            sl = pl.ds(v * 64 + k * LANES, LANES)
                            xv = qbuf_v[b, sl]
                            hi, lo = _key_full(plsc.bitcast(xv, jnp.int32))
                            g0 = plsc.load_gather(hist_v, [hi]).astype(jnp.float32)
                            g1 = plsc.load_gather(hist_v, [hi + 1]).astype(jnp.float32)
                            rank = g0 + (g1 - g0) * (lo.astype(jnp.float32) * inv_bin)
                            obuf_v[b, sl] = rank * inv_n
                        return 0
                    lax.fori_loop(0, QCHUNK // 64, vec_body, 0)
                    pltpu.async_copy(
                        obuf_v.at[b],
                        rk_hbm.at[row, pl.ds(ch * QCHUNK, QCHUNK)], so[b])
                return 0
            lax.fori_loop(0, NQCHUNK // 2, q_pair, 0)
            for b in range(2):
                pltpu.make_async_copy(
                    obuf_v.at[b],
                    rk_hbm.at[row, pl.ds((NQCHUNK - 2 + b) * QCHUNK, QCHUNK)],
                    so[b]).wait()
            return cnt

        cnt = jnp.zeros((LANES,), jnp.int32)
        cnt = row_pass(0, cnt)
        cnt = row_pass(1, cnt)
        cnt_v[...] = cnt
        pltpu.sync_copy(cnt_v, cnt_hbm.at[wid])

    @functools.partial(
        pl.kernel,
        out_type=jax.ShapeDtypeStruct((NW, NBINS), jnp.int32),
        mesh=mesh,
        scratch_types=[
            pltpu.VMEM((NBINS,), jnp.int32),
            pltpu.VMEM((CHUNK,), jnp.float32),
            pltpu.VMEM((LANES,), jnp.int32),
        ],
        interpret=interpret,
        compiler_params=pltpu.CompilerParams(needs_layout_passes=False),
    )
    def lohist_kernel(x_hbm, bstar_hbm, lh_hbm, hist_v, buf_v, b_v):
        wid = _wid()
        pltpu.sync_copy(bstar_hbm, b_v)
        bstar = b_v[...]
        ones = jnp.ones((LANES,), jnp.int32)

        def zero_body(i, _):
            hist_v[pl.ds(i * LANES, LANES)] = jnp.zeros((LANES,), jnp.int32)
            return 0
        lax.fori_loop(0, NBINS // LANES, zero_body, 0)

        def row_pass(j, _):
            row = wid * ROWS_PER_W + j

            def chunk_body(c, _):
                pltpu.sync_copy(x_hbm.at[row, pl.ds(c * CHUNK, CHUNK)], buf_v)

                def vec_body(v, _):
                    xv = buf_v[pl.ds(v * LANES, LANES)]
                    hi, lo = _key_full(plsc.bitcast(xv, jnp.int32))
                    plsc.addupdate_scatter(hist_v, [lo], ones, mask=hi == bstar)
                    return 0
                lax.fori_loop(0, CHUNK // LANES, vec_body, 0)
                return 0
            lax.fori_loop(0, NCHUNK, chunk_body, 0)
            return 0

        row_pass(0, 0)
        row_pass(1, 0)
        pltpu.sync_copy(hist_v, lh_hbm.at[wid])

    return main_kernel, lohist_kernel


_main_kernel, _lohist_kernel = _make_kernels()


def _exact_eps(xf, cdf):
    gcum = jnp.sum(cdf[:, : NBINS + 1], axis=0)
    bstar = jnp.searchsorted(gcum, K_IDX, side="right").astype(jnp.int32) - 1
    rstar = K_IDX - gcum[bstar]
    lh = jnp.sum(_lohist_kernel(xf, jnp.full((LANES,), bstar, jnp.int32)), axis=0)
    lc = jnp.cumsum(lh)
    lostar = jnp.searchsorted(lc, rstar, side="right").astype(jnp.int32)
    key = (bstar << 16) | lostar
    orig = jnp.where(key < 0, key ^ jnp.int32(_INT_MIN), ~key)
    eps = lax.bitcast_convert_type(orig, jnp.float32)
    return jnp.maximum(eps, jnp.float32(CLIP_LO))


def kernel(x):
    xf = x.reshape(BATCH, NUMEL)
    h_idx = (jnp.arange(OUT_H) * IN_H) // OUT_H
    w_idx = (jnp.arange(OUT_W) * IN_W) // OUT_W
    flat_idx = (h_idx[:, None] * IN_W + w_idx[None, :]).reshape(QN)
    q = jnp.take(xf, flat_idx, axis=1)

    ranks, cdf, counts = _main_kernel(xf, q)
    c = jnp.sum(counts)
    t = lax.cond(c > K_IDX, lambda: jnp.float32(CLIP_LO), lambda: _exact_eps(xf, cdf))

    out = jnp.where(q > t, ranks, jnp.float32(0.0))
    return out.reshape(BATCH, OUT_H, OUT_W)


# speedup vs baseline: 595.1412x; 2.1066x over previous
import functools

import jax
import jax.numpy as jnp
from jax import lax
from jax.experimental import pallas as pl
from jax.experimental.pallas import tpu as pltpu
from jax.experimental.pallas import tpu_sc as plsc

IN_H = IN_W = 512
OUT_H = OUT_W = 224
BATCH = 64
NUMEL = IN_H * IN_W
TOTAL = BATCH * NUMEL
K_IDX = int(0.1 * TOTAL)
CLIP_LO = 1e-3
NBINS = 1 << 16
HPAD = NBINS + 16
QN = OUT_H * OUT_W

NC, NS, LANES = 2, 16, 16
NW = NC * NS
ROWS_PER_W = BATCH // NW
CHUNK = 8192
NCHUNK = NUMEL // CHUNK
QCHUNK = 6272
NQCHUNK = QN // QCHUNK
NSEG = 16
SEGBINS = NBINS // NSEG

_INT_MIN = -2147483648


def _wid():
    return lax.axis_index("s") * NC + lax.axis_index("c")


def _key_full(ib):
    key = ib ^ ((ib >> 31) | jnp.int32(_INT_MIN))
    hi = (key >> 16) & jnp.int32(0xFFFF)
    lo = key & jnp.int32(0xFFFF)
    return hi, lo


def _key_hi(ib):
    s = ib >> 16
    return s ^ ((s >> 31) | jnp.int32(0x8000))


def _make_kernels(interpret=False):
    mesh = plsc.VectorSubcoreMesh(core_axis_name="c", subcore_axis_name="s")

    @functools.partial(
        pl.kernel,
        out_type=[
            jax.ShapeDtypeStruct((BATCH, QN), jnp.float32),
            jax.ShapeDtypeStruct((NW, LANES), jnp.int32),
        ],
        mesh=mesh,
        scratch_types=[
            pltpu.VMEM((HPAD,), jnp.int32),
            pltpu.VMEM((2, CHUNK), jnp.float32),
            pltpu.VMEM((2, QCHUNK), jnp.float32),
            pltpu.VMEM((2, QCHUNK), jnp.float32),
            pltpu.VMEM((LANES,), jnp.int32),
            pltpu.SemaphoreType.DMA,
            pltpu.SemaphoreType.DMA,
            pltpu.SemaphoreType.DMA,
            pltpu.SemaphoreType.DMA,
            pltpu.SemaphoreType.DMA,
            pltpu.SemaphoreType.DMA,
        ],
        interpret=interpret,
        compiler_params=pltpu.CompilerParams(needs_layout_passes=False),
    )
    def main_kernel(x_hbm, q_hbm, rk_hbm, cnt_hbm,
                    hist_v, xbuf_v, qbuf_v, obuf_v, cnt_v,
                    sx0, sx1, sq0, sq1, so0, so1):
        wid = _wid()
        sx = (sx0, sx1)
        sq = (sq0, sq1)
        so = (so0, so1)
        ones = jnp.ones((LANES,), jnp.int32)
        zeros = jnp.zeros((LANES,), jnp.int32)
        clip = jnp.float32(CLIP_LO)
        inv_bin = jnp.float32(1.0 / 65536.0)
        inv_n = jnp.float32(1.0 / NUMEL)

        def row_pass(j, cnt):
            row = wid * ROWS_PER_W + j

            def zero_body(i):
                hist_v[pl.ds(i, LANES)] = zeros
            plsc.parallel_loop(0, NBINS, LANES, unroll=8)(zero_body)
            hist_v[pl.ds(NBINS, LANES)] = zeros

            pltpu.async_copy(x_hbm.at[row, pl.ds(0, CHUNK)], xbuf_v.at[0], sx[0])

            def chunk_pair(cc, cnt):
                for b in range(2):
                    ch = cc * 2 + b
                    pltpu.make_async_copy(
                        x_hbm.at[row, pl.ds(ch * CHUNK, CHUNK)],
                        xbuf_v.at[b], sx[b]).wait()
                    if b == 0:
                        pltpu.async_copy(
                            x_hbm.at[row, pl.ds((ch + 1) * CHUNK, CHUNK)],
                            xbuf_v.at[1], sx[1])
                    else:
                        @pl.when(cc < NCHUNK // 2 - 1)
                        def _():
                            pltpu.async_copy(
                                x_hbm.at[row, pl.ds((ch + 1) * CHUNK, CHUNK)],
                                xbuf_v.at[0], sx[0])

                    def scatter_body(v, cnt):
                        xv = xbuf_v[b, pl.ds(v, LANES)]
                        hi = _key_hi(plsc.bitcast(xv, jnp.int32))
                        plsc.addupdate_scatter(hist_v, [hi], ones)
                        return cnt + jnp.where(xv <= clip, 1, 0).astype(jnp.int32)
                    cnt = plsc.parallel_loop(
                        0, CHUNK, LANES, unroll=8, carry=cnt)(scatter_body)
                return cnt
            cnt = lax.fori_loop(0, NCHUNK // 2, chunk_pair, cnt)

            def acc_body(i, accs):
                return tuple(
                    accs[s] + hist_v[pl.ds(s * SEGBINS + i, LANES)]
                    for s in range(NSEG))
            accs = plsc.parallel_loop(
                0, SEGBINS, LANES, unroll=2,
                carry=(zeros,) * NSEG)(acc_body)
            seg_tot = [jnp.sum(a) for a in accs]

            seg_base = []
            run = jnp.int32(0)
            for s in range(NSEG):
                seg_base.append(run)
                run = run + seg_tot[s]

            def cum_body(i, runs):
                new_runs = []
                for s in range(NSEG):
                    off = s * SEGBINS + i
                    h = hist_v[pl.ds(off, LANES)]
                    hist_v[pl.ds(off, LANES)] = plsc.cumsum(h) - h + runs[s]
                    new_runs.append(runs[s] + jnp.sum(h))
                return tuple(new_runs)
            plsc.parallel_loop(
                0, SEGBINS, LANES, unroll=2, carry=tuple(seg_base))(cum_body)
            hist_v[pl.ds(NBINS, LANES)] = zeros + run

            pltpu.async_copy(q_hbm.at[row, pl.ds(0, QCHUNK)], qbuf_v.at[0], sq[0])

            def q_pair(cc, _):
                for b in range(2):
                    ch = cc * 2 + b
                    pltpu.make_async_copy(
                        q_hbm.at[row, pl.ds(ch * QCHUNK, QCHUNK)],
                        qbuf_v.at[b], sq[b]).wait()
                    if b == 0:
                        pltpu.async_copy(
                            q_hbm.at[row, pl.ds((ch + 1) * QCHUNK, QCHUNK)],
                            qbuf_v.at[1], sq[1])
                    else:
                        @pl.when(cc < NQCHUNK // 2 - 1)
                        def _():
                            pltpu.async_copy(
                                q_hbm.at[row, pl.ds((ch + 1) * QCHUNK, QCHUNK)],
                                qbuf_v.at[0], sq[0])

                    @pl.when(cc > 0)
                    def _():
                        pltpu.make_async_copy(
                            obuf_v.at[b],
                            rk_hbm.at[row, pl.ds((ch - 2) * QCHUNK, QCHUNK)],
                            so[b]).wait()

                    def rank_body(v):
                        sl = pl.ds(v, LANES)
                        xv = qbuf_v[b, sl]
                        hi, lo = _key_full(plsc.bitcast(xv, jnp.int32))
                        g0 = plsc.load_gather(hist_v, [hi]).astype(jnp.float32)
                        g1 = plsc.load_gather(hist_v, [hi + 1]).astype(jnp.float32)
                        rank = g0 + (g1 - g0) * (lo.astype(jnp.float32) * inv_bin)
                        obuf_v[b, sl] = jnp.where(
                            xv > clip, rank * inv_n, jnp.float32(0.0))
                    plsc.parallel_loop(0, QCHUNK, LANES, unroll=4)(rank_body)
                    pltpu.async_copy(
                        obuf_v.at[b],
                        rk_hbm.at[row, pl.ds(ch * QCHUNK, QCHUNK)], so[b])
                return 0
            lax.fori_loop(0, NQCHUNK // 2, q_pair, 0)
            for b in range(2):
                pltpu.make_async_copy(
                    obuf_v.at[b],
                    rk_hbm.at[row, pl.ds((NQCHUNK - 2 + b) * QCHUNK, QCHUNK)],
                    so[b]).wait()
            return cnt

        cnt = jnp.zeros((LANES,), jnp.int32)
        cnt = row_pass(0, cnt)
        cnt = row_pass(1, cnt)
        cnt_v[...] = cnt
        pltpu.sync_copy(cnt_v, cnt_hbm.at[wid])

    @functools.partial(
        pl.kernel,
        out_type=jax.ShapeDtypeStruct((NW, NBINS), jnp.int32),
        mesh=mesh,
        scratch_types=[
            pltpu.VMEM((NBINS,), jnp.int32),
            pltpu.VMEM((CHUNK,), jnp.float32),
        ],
        interpret=interpret,
        compiler_params=pltpu.CompilerParams(needs_layout_passes=False),
    )
    def hihist_kernel(x_hbm, hh_hbm, hist_v, buf_v):
        wid = _wid()
        ones = jnp.ones((LANES,), jnp.int32)

        def zero_body(i):
            hist_v[pl.ds(i, LANES)] = jnp.zeros((LANES,), jnp.int32)
        plsc.parallel_loop(0, NBINS, LANES, unroll=8)(zero_body)

        for j in range(ROWS_PER_W):
            row = wid * ROWS_PER_W + j

            def chunk_body(c, _):
                pltpu.sync_copy(x_hbm.at[row, pl.ds(c * CHUNK, CHUNK)], buf_v)

                def vec_body(v):
                    xv = buf_v[pl.ds(v, LANES)]
                    hi = _key_hi(plsc.bitcast(xv, jnp.int32))
                    plsc.addupdate_scatter(hist_v, [hi], ones)
                plsc.parallel_loop(0, CHUNK, LANES, unroll=4)(vec_body)
                return 0
            lax.fori_loop(0, NCHUNK, chunk_body, 0)
        pltpu.sync_copy(hist_v, hh_hbm.at[wid])

    @functools.partial(
        pl.kernel,
        out_type=jax.ShapeDtypeStruct((NW, NBINS), jnp.int32),
        mesh=mesh,
        scratch_types=[
            pltpu.VMEM((NBINS,), jnp.int32),
            pltpu.VMEM((CHUNK,), jnp.float32),
            pltpu.VMEM((LANES,), jnp.int32),
        ],
        interpret=interpret,
        compiler_params=pltpu.CompilerParams(needs_layout_passes=False),
    )
    def lohist_kernel(x_hbm, bstar_hbm, lh_hbm, hist_v, buf_v, b_v):
        wid = _wid()
        pltpu.sync_copy(bstar_hbm, b_v)
        bstar = b_v[...]
        ones = jnp.ones((LANES,), jnp.int32)

        def zero_body(i):
            hist_v[pl.ds(i, LANES)] = jnp.zeros((LANES,), jnp.int32)
        plsc.parallel_loop(0, NBINS, LANES, unroll=8)(zero_body)

        for j in range(ROWS_PER_W):
            row = wid * ROWS_PER_W + j

            def chunk_body(c, _):
                pltpu.sync_copy(x_hbm.at[row, pl.ds(c * CHUNK, CHUNK)], buf_v)

                def vec_body(v):
                    xv = buf_v[pl.ds(v, LANES)]
                    hi, lo = _key_full(plsc.bitcast(xv, jnp.int32))
                    plsc.addupdate_scatter(hist_v, [lo], ones, mask=hi == bstar)
                plsc.parallel_loop(0, CHUNK, LANES, unroll=4)(vec_body)
                return 0
            lax.fori_loop(0, NCHUNK, chunk_body, 0)
        pltpu.sync_copy(hist_v, lh_hbm.at[wid])

    return main_kernel, hihist_kernel, lohist_kernel


_main_kernel, _hihist_kernel, _lohist_kernel = _make_kernels()


def _regate(xf, q, ranks):
    hh = jnp.sum(_hihist_kernel(xf), axis=0)
    gcum = jnp.concatenate([jnp.zeros((1,), jnp.int32), jnp.cumsum(hh)])
    bstar = jnp.searchsorted(gcum, K_IDX, side="right").astype(jnp.int32) - 1
    rstar = K_IDX - gcum[bstar]
    lh = jnp.sum(_lohist_kernel(xf, jnp.full((LANES,), bstar, jnp.int32)), axis=0)
    lc = jnp.cumsum(lh)
    lostar = jnp.searchsorted(lc, rstar, side="right").astype(jnp.int32)
    key = (bstar << 16) | lostar
    orig = jnp.where(key < 0, key ^ jnp.int32(_INT_MIN), ~key)
    eps = lax.bitcast_convert_type(orig, jnp.float32)
    t = jnp.maximum(eps, jnp.float32(CLIP_LO))
    return jnp.where(q > t, ranks, jnp.float32(0.0))


def kernel(x):
    xf = x.reshape(BATCH, NUMEL)
    h_idx = (jnp.arange(OUT_H) * IN_H) // OUT_H
    w_idx = (jnp.arange(OUT_W) * IN_W) // OUT_W
    flat_idx = (h_idx[:, None] * IN_W + w_idx[None, :]).reshape(QN)
    q = jnp.take(xf, flat_idx, axis=1)

    ranks, counts = _main_kernel(xf, q)
    c = jnp.sum(counts)
    out = lax.cond(c > K_IDX, lambda: ranks, lambda: _regate(xf, q, ranks))
    return out.reshape(BATCH, OUT_H, OUT_W)
